# baseline (speedup 1.0000x reference)
import jax
import jax.numpy as jnp
import numpy as np
from jax.experimental import pallas as pl

_F32 = jnp.float32
_BIAS = float(np.log(np.expm1(1.0)))


def _adj(edge_ref, n_src, n_dst):
    e = edge_ref[...]
    src = e[0, :]
    dst = e[1, :]
    n_e = src.shape[0]
    s_oh = (src[:, None] == jax.lax.broadcasted_iota(jnp.int32, (n_e, n_src), 1)
            ).astype(_F32)
    d_oh = (dst[:, None] == jax.lax.broadcasted_iota(jnp.int32, (n_e, n_dst), 1)
            ).astype(_F32)
    return jax.lax.dot_general(
        d_oh, s_oh, (((0,), (0,)), ((), ())), preferred_element_type=_F32)


def _mm(a, b):
    return jax.lax.dot_general(
        a, b, (((1,), (0,)), ((), ())), preferred_element_type=_F32)


def _body(ei_tj, ei_jt, ei_jj, x_joint, x_torso, Wj, bj, Wt, bt,
          W1_tj_rel, b1_tj, W1_tj_root, W1_jj_rel, b1_jj, W1_jj_root,
          W1_jt_rel, b1_jt, W1_jt_root,
          W2_tj_rel, b2_tj, W2_tj_root, W2_jj_rel, b2_jj, W2_jj_root,
          Wbig, bbig, loc_ref, scale_ref):
    h_j = _mm(x_joint[...], Wj[...]) + bj[...][None, :]
    h_t = _mm(x_torso[...], Wt[...]) + bt[...][None, :]

    A_tj = _adj(ei_tj, 10, 80)
    A_jt = _adj(ei_jt, 80, 10)
    A_jj = _adj(ei_jj, 80, 80)

    x1j = jnp.concatenate([_mm(A_tj, h_t), _mm(A_jj, h_j), h_j], axis=1)
    w1j = jnp.concatenate(
        [W1_tj_rel[...], W1_jj_rel[...], W1_tj_root[...] + W1_jj_root[...]],
        axis=0)
    j1 = jnp.tanh(_mm(x1j, w1j) + (b1_tj[...] + b1_jj[...])[None, :])
    x1t = jnp.concatenate([_mm(A_jt, h_j), h_t], axis=1)
    w1t = jnp.concatenate([W1_jt_rel[...], W1_jt_root[...]], axis=0)
    t1 = jnp.tanh(_mm(x1t, w1t) + b1_jt[...][None, :])

    x2j = jnp.concatenate([_mm(A_tj, t1), _mm(A_jj, j1), j1], axis=1)
    w2j = jnp.concatenate(
        [W2_tj_rel[...], W2_jj_rel[...], W2_tj_root[...] + W2_jj_root[...]],
        axis=0)
    j2 = jnp.tanh(_mm(x2j, w2j) + (b2_tj[...] + b2_jj[...])[None, :])

    out16 = _mm(j2, Wbig[...]) + bbig[...][None, :]
    col = jax.lax.broadcasted_iota(jnp.int32, (80, 16), 1)
    head2 = 2 * (jax.lax.broadcasted_iota(jnp.int32, (80, 16), 0) % 8)
    loc = jnp.sum(jnp.where(col == head2, out16, 0.0), axis=1)
    pre = jnp.sum(jnp.where(col == head2 + 1, out16, 0.0), axis=1)
    scale = jnp.maximum(jax.nn.softplus(pre + _BIAS), 1e-4)
    loc_ref[...] = loc.reshape(10, 8)
    scale_ref[...] = scale.reshape(10, 8)


def kernel(x_joint, x_torso, Wj, bj, Wt, bt,
           W1_tj_rel, b1_tj, W1_tj_root, W1_jj_rel, b1_jj, W1_jj_root,
           W1_jt_rel, b1_jt, W1_jt_root,
           W2_tj_rel, b2_tj, W2_tj_root, W2_jj_rel, b2_jj, W2_jj_root,
           W2_jt_rel, b2_jt, W2_jt_root,
           Wout, bout, ei_tj, ei_jt, ei_jj):
    Wbig = jnp.transpose(Wout, (1, 0, 2)).reshape(64, 16)
    bbig = bout.reshape(16)
    loc, scale = pl.pallas_call(
        _body,
        out_shape=(jax.ShapeDtypeStruct((10, 8), _F32),
                   jax.ShapeDtypeStruct((10, 8), _F32)),
    )(ei_tj.astype(jnp.int32), ei_jt.astype(jnp.int32),
      ei_jj.astype(jnp.int32),
      x_joint, x_torso, Wj, bj, Wt, bt,
      W1_tj_rel, b1_tj, W1_tj_root, W1_jj_rel, b1_jj, W1_jj_root,
      W1_jt_rel, b1_jt, W1_jt_root,
      W2_tj_rel, b2_tj, W2_tj_root, W2_jj_rel, b2_jj, W2_jj_root,
      Wbig, bbig)
    return (loc, scale)


# speedup vs baseline: 1.0196x; 1.0196x over previous
import jax
import jax.numpy as jnp
import numpy as np
from jax.experimental import pallas as pl

_F32 = jnp.float32
_BIAS = float(np.log(np.expm1(1.0)))


def _adj(edge_ref, n_src, n_dst):
    e = edge_ref[...]
    src = e[0, :]
    dst = e[1, :]
    n_e = src.shape[0]
    s_oh = (src[:, None] == jax.lax.broadcasted_iota(jnp.int32, (n_e, n_src), 1)
            ).astype(jnp.bfloat16)
    d_oh = (dst[:, None] == jax.lax.broadcasted_iota(jnp.int32, (n_e, n_dst), 1)
            ).astype(jnp.bfloat16)
    return jax.lax.dot_general(
        d_oh, s_oh, (((0,), (0,)), ((), ())), preferred_element_type=_F32)


def _mm(a, b):
    return jax.lax.dot_general(
        a, b, (((1,), (0,)), ((), ())), preferred_element_type=_F32)


def _body(ei_tj, ei_jt, ei_jj, x_joint, x_torso, Wj, bj, Wt, bt,
          W1_tj_rel, b1_tj, W1_tj_root, W1_jj_rel, b1_jj, W1_jj_root,
          W1_jt_rel, b1_jt, W1_jt_root,
          W2_tj_rel, b2_tj, W2_tj_root, W2_jj_rel, b2_jj, W2_jj_root,
          Wbig, bbig, loc_ref, scale_ref):
    h_j = _mm(x_joint[...], Wj[...]) + bj[...][None, :]
    h_t = _mm(x_torso[...], Wt[...]) + bt[...][None, :]

    A_tj = _adj(ei_tj, 10, 80)
    A_jt = _adj(ei_jt, 80, 10)
    A_jj = _adj(ei_jj, 80, 80)

    x1j = jnp.concatenate([_mm(A_tj, h_t), _mm(A_jj, h_j), h_j], axis=1)
    w1j = jnp.concatenate(
        [W1_tj_rel[...], W1_jj_rel[...], W1_tj_root[...] + W1_jj_root[...]],
        axis=0)
    j1 = jnp.tanh(_mm(x1j, w1j) + (b1_tj[...] + b1_jj[...])[None, :])
    x1t = jnp.concatenate([_mm(A_jt, h_j), h_t], axis=1)
    w1t = jnp.concatenate([W1_jt_rel[...], W1_jt_root[...]], axis=0)
    t1 = jnp.tanh(_mm(x1t, w1t) + b1_jt[...][None, :])

    x2j = jnp.concatenate([_mm(A_tj, t1), _mm(A_jj, j1), j1], axis=1)
    w2j = jnp.concatenate(
        [W2_tj_rel[...], W2_jj_rel[...], W2_tj_root[...] + W2_jj_root[...]],
        axis=0)
    j2 = jnp.tanh(_mm(x2j, w2j) + (b2_tj[...] + b2_jj[...])[None, :])

    out16 = _mm(j2, Wbig[...]) + bbig[...][None, :]
    col = jax.lax.broadcasted_iota(jnp.int32, (80, 16), 1)
    head2 = 2 * (jax.lax.broadcasted_iota(jnp.int32, (80, 16), 0) % 8)
    loc = jnp.sum(jnp.where(col == head2, out16, 0.0), axis=1)
    pre = jnp.sum(jnp.where(col == head2 + 1, out16, 0.0), axis=1)
    scale = jnp.maximum(jax.nn.softplus(pre + _BIAS), 1e-4)
    loc_ref[...] = loc.reshape(10, 8)
    scale_ref[...] = scale.reshape(10, 8)


def kernel(x_joint, x_torso, Wj, bj, Wt, bt,
           W1_tj_rel, b1_tj, W1_tj_root, W1_jj_rel, b1_jj, W1_jj_root,
           W1_jt_rel, b1_jt, W1_jt_root,
           W2_tj_rel, b2_tj, W2_tj_root, W2_jj_rel, b2_jj, W2_jj_root,
           W2_jt_rel, b2_jt, W2_jt_root,
           Wout, bout, ei_tj, ei_jt, ei_jj):
    Wbig = jnp.transpose(Wout, (1, 0, 2)).reshape(64, 16)
    bbig = bout.reshape(16)
    loc, scale = pl.pallas_call(
        _body,
        out_shape=(jax.ShapeDtypeStruct((10, 8), _F32),
                   jax.ShapeDtypeStruct((10, 8), _F32)),
    )(ei_tj.astype(jnp.int32), ei_jt.astype(jnp.int32),
      ei_jj.astype(jnp.int32),
      x_joint, x_torso, Wj, bj, Wt, bt,
      W1_tj_rel, b1_tj, W1_tj_root, W1_jj_rel, b1_jj, W1_jj_root,
      W1_jt_rel, b1_jt, W1_jt_root,
      W2_tj_rel, b2_tj, W2_tj_root, W2_jj_rel, b2_jj, W2_jj_root,
      Wbig, bbig)
    return (loc, scale)
